# Initial kernel scaffold; baseline (speedup 1.0000x reference)
import jax
import jax.numpy as jnp
from jax.experimental import pallas as pl


def kernel(x, edge_index, batch, Wm, bm, Wu, bu, We, be):
    raise NotImplementedError("write your pallas kernel here")


# speedup vs baseline: 4.9957x; 4.9957x over previous
import functools

import jax
import jax.numpy as jnp
from jax import lax
from jax.experimental import pallas as pl
from jax.experimental.pallas import tpu as pltpu
from jax.experimental.pallas import tpu_sc as plsc

NC = 2
NS = 16
LANES = 16


def _sc_edge_pass(msg, src, dst):
  n_pad, s = msg.shape
  e = src.shape[0]
  per_tile = e // (NC * NS)
  K = 80
  chunks = per_tile // K
  acc_per_tile = n_pad // NS
  ZR = 64

  mesh = plsc.VectorSubcoreMesh(core_axis_name="c", subcore_axis_name="s")

  @functools.partial(
      pl.kernel,
      out_type=jax.ShapeDtypeStruct((NC * n_pad, s), jnp.float32),
      mesh=mesh,
      scratch_types=[
          pltpu.VMEM((K,), jnp.int32),
          pltpu.VMEM((K,), jnp.int32),
          pltpu.VMEM((K, s), jnp.float32),
          pltpu.VMEM((ZR, s), jnp.float32),
          pltpu.VMEM_SHARED((n_pad, s), jnp.float32),
      ],
  )
  def kern(msg_hbm, src_hbm, dst_hbm, out_hbm, srcb, dstb, rows, zbuf, agg):
    c = lax.axis_index("c")
    t = lax.axis_index("s")

    zero = jnp.zeros((LANES,), jnp.float32)
    for i in range(ZR):
      for j in range(s // LANES):
        zbuf[i, pl.ds(j * LANES, LANES)] = zero
    astripe = t * acc_per_tile
    for i in range(acc_per_tile // ZR):
      pltpu.sync_copy(zbuf, agg.at[pl.ds(astripe + i * ZR, ZR)])
    plsc.subcore_barrier()

    base = (c * NS + t) * per_tile

    def body(j, carry):
      off = base + j * K
      pltpu.sync_copy(src_hbm.at[pl.ds(off, K)], srcb)
      pltpu.sync_copy(dst_hbm.at[pl.ds(off, K)], dstb)
      pltpu.sync_copy(msg_hbm.at[srcb], rows)
      pltpu.sync_copy(rows, agg.at[dstb], add=True)
      return carry

    lax.fori_loop(0, chunks, body, 0)
    plsc.subcore_barrier()

    for i in range(acc_per_tile // ZR):
      r0 = astripe + i * ZR
      pltpu.sync_copy(agg.at[pl.ds(r0, ZR)], zbuf)
      pltpu.sync_copy(zbuf, out_hbm.at[pl.ds(c * n_pad + r0, ZR)])

  return kern(msg, src, dst)


_BLK = 1024


def _row_spec(s):
  return pl.BlockSpec((_BLK, s), lambda i: (i, 0))


def _full_spec(a, b):
  return pl.BlockSpec((a, b), lambda i: (0, 0))


def _dot_t(x, w):
  return lax.dot_general(x, w, (((1,), (1,)), ((), ())),
                         preferred_element_type=jnp.float32)


def _agg_specs(n, s):
  nb = n // _BLK
  return [pl.BlockSpec((_BLK, s), lambda i: (i, 0)),
          pl.BlockSpec((_BLK, s), lambda i: (i + nb, 0))]


def _tc_msg0(bm0, n, s):
  def body(bm_ref, o_ref):
    o_ref[...] = jnp.broadcast_to(jnp.maximum(bm_ref[...], 0.0), (_BLK, s))

  return pl.pallas_call(
      body,
      grid=(n // _BLK,),
      in_specs=[_full_spec(1, s)],
      out_specs=_row_spec(s),
      out_shape=jax.ShapeDtypeStruct((n, s), jnp.float32),
  )(bm0.reshape(1, s))


def _tc_boundary(state, agg2, wu, bu, wm, bm):
  n, s = state.shape

  def body(st_ref, a0_ref, a1_ref, wu_ref, bu_ref, wm_ref, bm_ref,
           snew_ref, msg_ref):
    agg = a0_ref[...] + a1_ref[...]
    upd = jnp.maximum(_dot_t(agg, wu_ref[...]) + bu_ref[...], 0.0)
    snew = st_ref[...] + upd
    snew_ref[...] = snew
    msg_ref[...] = jnp.maximum(_dot_t(snew, wm_ref[...]) + bm_ref[...], 0.0)

  a0_spec, a1_spec = _agg_specs(n, s)
  return pl.pallas_call(
      body,
      grid=(n // _BLK,),
      in_specs=[
          _row_spec(s),
          a0_spec,
          a1_spec,
          _full_spec(s, s),
          _full_spec(1, s),
          _full_spec(s, s),
          _full_spec(1, s),
      ],
      out_specs=[_row_spec(s), _row_spec(s)],
      out_shape=[jax.ShapeDtypeStruct((n, s), jnp.float32),
                 jax.ShapeDtypeStruct((n, s), jnp.float32)],
  )(state, agg2, agg2, wu, bu.reshape(1, s), wm, bm.reshape(1, s))


def _tc_final(state, agg2, wu, bu, we, be):
  n, s = state.shape
  two_m = we.shape[0]
  m = two_m // 2

  def body(st_ref, a0_ref, a1_ref, wu_ref, bu_ref, we_ref, be_ref, o_ref):
    agg = a0_ref[...] + a1_ref[...]
    upd = jnp.maximum(_dot_t(agg, wu_ref[...]) + bu_ref[...], 0.0)
    snew = st_ref[...] + upd
    out = _dot_t(snew, we_ref[...]) + be_ref[...]
    col = lax.broadcasted_iota(jnp.int32, (_BLK, two_m), 1)
    o_ref[...] = jnp.where(col < m, out, jnp.exp(out))

  a0_spec, a1_spec = _agg_specs(n, s)
  return pl.pallas_call(
      body,
      grid=(n // _BLK,),
      in_specs=[
          _row_spec(s),
          a0_spec,
          a1_spec,
          _full_spec(s, s),
          _full_spec(1, s),
          _full_spec(two_m, s),
          _full_spec(1, two_m),
      ],
      out_specs=_row_spec(two_m),
      out_shape=jax.ShapeDtypeStruct((n, two_m), jnp.float32),
  )(state, agg2, agg2, wu, bu.reshape(1, s), we,
    be.reshape(1, two_m))


def kernel(x, edge_index, batch, Wm, bm, Wu, bu, We, be):
  n = x.shape[0]
  s = Wm.shape[1]
  n_pad = ((n + _BLK - 1) // _BLK) * _BLK
  src = edge_index[0]
  dst = edge_index[1]

  msg = _tc_msg0(bm[0], n_pad, s)
  state = jnp.zeros((n_pad, s), dtype=jnp.float32)
  for r in range(Wm.shape[0]):
    agg2 = _sc_edge_pass(msg, src, dst)
    if r + 1 < Wm.shape[0]:
      state, msg = _tc_boundary(state, agg2, Wu[r], bu[r], Wm[r + 1],
                                bm[r + 1])
    else:
      out = _tc_final(state, agg2, Wu[r], bu[r], We, be)
  return out[:n].astype(x.dtype)


# speedup vs baseline: 8.0108x; 1.6035x over previous
import functools

import jax
import jax.numpy as jnp
from jax import lax
from jax.experimental import pallas as pl
from jax.experimental.pallas import tpu as pltpu
from jax.experimental.pallas import tpu_sc as plsc

NC = 2
NS = 16
LANES = 16


def _sc_edge_pass(msg, src, dst):
  n_pad, s = msg.shape
  e = src.shape[0]
  per_tile = e // (NC * NS)
  K = 80
  chunks = per_tile // K
  acc_per_tile = n_pad // NS
  ZR = 64

  mesh = plsc.VectorSubcoreMesh(core_axis_name="c", subcore_axis_name="s")

  @functools.partial(
      pl.kernel,
      out_type=jax.ShapeDtypeStruct((NC * n_pad, s), jnp.float32),
      mesh=mesh,
      scratch_types=[
          pltpu.VMEM((2, K), jnp.int32),
          pltpu.VMEM((2, K), jnp.int32),
          pltpu.VMEM((2, K, s), jnp.float32),
          pltpu.VMEM((ZR, s), jnp.float32),
          pltpu.VMEM_SHARED((n_pad, s), jnp.float32),
          pltpu.SemaphoreType.DMA,
          pltpu.SemaphoreType.DMA,
      ],
  )
  def kern(msg_hbm, src_hbm, dst_hbm, out_hbm, srcb, dstb, rows, zbuf, agg,
           sem0, sem1):
    c = lax.axis_index("c")
    t = lax.axis_index("s")
    sems = (sem0, sem1)

    zero = jnp.zeros((LANES,), jnp.float32)
    for i in range(ZR):
      for j in range(s // LANES):
        zbuf[i, pl.ds(j * LANES, LANES)] = zero
    astripe = t * acc_per_tile
    for i in range(acc_per_tile // ZR):
      pltpu.sync_copy(zbuf, agg.at[pl.ds(astripe + i * ZR, ZR)])
    plsc.subcore_barrier()

    base = (c * NS + t) * per_tile

    def load_and_fire(j, b):
      off = base + j * K
      pltpu.sync_copy(src_hbm.at[pl.ds(off, K)], srcb.at[b])
      pltpu.sync_copy(dst_hbm.at[pl.ds(off, K)], dstb.at[b])
      pltpu.async_copy(msg_hbm.at[srcb.at[b]], rows.at[b], sems[b])

    def drain_and_scatter(b):
      pltpu.make_async_copy(msg_hbm.at[srcb.at[b]], rows.at[b],
                            sems[b]).wait()
      pltpu.sync_copy(rows.at[b], agg.at[dstb.at[b]], add=True)

    for b in range(min(2, chunks)):
      load_and_fire(b, b)

    def body(g, carry):
      for b in range(2):
        j = 2 * g + b
        drain_and_scatter(b)

        @pl.when(j + 2 < chunks)
        def _():
          load_and_fire(j + 2, b)

      return carry

    pairs = (chunks - 1) // 2
    lax.fori_loop(0, pairs, body, 0)
    for k in range(chunks - 2 * pairs):
      drain_and_scatter((2 * pairs + k) % 2)
    plsc.subcore_barrier()

    for i in range(acc_per_tile // ZR):
      r0 = astripe + i * ZR
      pltpu.sync_copy(agg.at[pl.ds(r0, ZR)], zbuf)
      pltpu.sync_copy(zbuf, out_hbm.at[pl.ds(c * n_pad + r0, ZR)])

  return kern(msg, src, dst)


_BLK = 1024


def _row_spec(s):
  return pl.BlockSpec((_BLK, s), lambda i: (i, 0))


def _full_spec(a, b):
  return pl.BlockSpec((a, b), lambda i: (0, 0))


def _dot_t(x, w):
  return lax.dot_general(x, w, (((1,), (1,)), ((), ())),
                         preferred_element_type=jnp.float32)


def _agg_specs(n, s):
  nb = n // _BLK
  return [pl.BlockSpec((_BLK, s), lambda i: (i, 0)),
          pl.BlockSpec((_BLK, s), lambda i: (i + nb, 0))]


def _tc_msg0(bm0, n, s):
  def body(bm_ref, o_ref):
    o_ref[...] = jnp.broadcast_to(jnp.maximum(bm_ref[...], 0.0), (_BLK, s))

  return pl.pallas_call(
      body,
      grid=(n // _BLK,),
      in_specs=[_full_spec(1, s)],
      out_specs=_row_spec(s),
      out_shape=jax.ShapeDtypeStruct((n, s), jnp.float32),
  )(bm0.reshape(1, s))


def _tc_boundary(state, agg2, wu, bu, wm, bm):
  n, s = state.shape

  def body(st_ref, a0_ref, a1_ref, wu_ref, bu_ref, wm_ref, bm_ref,
           snew_ref, msg_ref):
    agg = a0_ref[...] + a1_ref[...]
    upd = jnp.maximum(_dot_t(agg, wu_ref[...]) + bu_ref[...], 0.0)
    snew = st_ref[...] + upd
    snew_ref[...] = snew
    msg_ref[...] = jnp.maximum(_dot_t(snew, wm_ref[...]) + bm_ref[...], 0.0)

  a0_spec, a1_spec = _agg_specs(n, s)
  return pl.pallas_call(
      body,
      grid=(n // _BLK,),
      in_specs=[
          _row_spec(s),
          a0_spec,
          a1_spec,
          _full_spec(s, s),
          _full_spec(1, s),
          _full_spec(s, s),
          _full_spec(1, s),
      ],
      out_specs=[_row_spec(s), _row_spec(s)],
      out_shape=[jax.ShapeDtypeStruct((n, s), jnp.float32),
                 jax.ShapeDtypeStruct((n, s), jnp.float32)],
  )(state, agg2, agg2, wu, bu.reshape(1, s), wm, bm.reshape(1, s))


def _tc_final(state, agg2, wu, bu, we, be):
  n, s = state.shape
  two_m = we.shape[0]
  m = two_m // 2

  def body(st_ref, a0_ref, a1_ref, wu_ref, bu_ref, we_ref, be_ref, o_ref):
    agg = a0_ref[...] + a1_ref[...]
    upd = jnp.maximum(_dot_t(agg, wu_ref[...]) + bu_ref[...], 0.0)
    snew = st_ref[...] + upd
    out = _dot_t(snew, we_ref[...]) + be_ref[...]
    col = lax.broadcasted_iota(jnp.int32, (_BLK, two_m), 1)
    o_ref[...] = jnp.where(col < m, out, jnp.exp(out))

  a0_spec, a1_spec = _agg_specs(n, s)
  return pl.pallas_call(
      body,
      grid=(n // _BLK,),
      in_specs=[
          _row_spec(s),
          a0_spec,
          a1_spec,
          _full_spec(s, s),
          _full_spec(1, s),
          _full_spec(two_m, s),
          _full_spec(1, two_m),
      ],
      out_specs=_row_spec(two_m),
      out_shape=jax.ShapeDtypeStruct((n, two_m), jnp.float32),
  )(state, agg2, agg2, wu, bu.reshape(1, s), we,
    be.reshape(1, two_m))


def kernel(x, edge_index, batch, Wm, bm, Wu, bu, We, be):
  n = x.shape[0]
  s = Wm.shape[1]
  n_pad = ((n + _BLK - 1) // _BLK) * _BLK
  src = edge_index[0]
  dst = edge_index[1]

  msg = _tc_msg0(bm[0], n_pad, s)
  state = jnp.zeros((n_pad, s), dtype=jnp.float32)
  for r in range(Wm.shape[0]):
    agg2 = _sc_edge_pass(msg, src, dst)
    if r + 1 < Wm.shape[0]:
      state, msg = _tc_boundary(state, agg2, Wu[r], bu[r], Wm[r + 1],
                                bm[r + 1])
    else:
      out = _tc_final(state, agg2, Wu[r], bu[r], We, be)
  return out[:n].astype(x.dtype)


# speedup vs baseline: 9.8872x; 1.2342x over previous
import functools

import jax
import jax.numpy as jnp
from jax import lax
from jax.experimental import pallas as pl
from jax.experimental.pallas import tpu as pltpu
from jax.experimental.pallas import tpu_sc as plsc

NC = 2
NS = 16
LANES = 16


def _sc_edge_pass(msg, src, dst):
  n_pad, s = msg.shape
  e = src.shape[0]
  per_tile = e // (NC * NS)
  K = 80
  chunks = per_tile // K
  acc_per_tile = n_pad // NS
  ZR = 64

  mesh = plsc.VectorSubcoreMesh(core_axis_name="c", subcore_axis_name="s")

  @functools.partial(
      pl.kernel,
      out_type=jax.ShapeDtypeStruct((NC * n_pad, s), jnp.float32),
      mesh=mesh,
      scratch_types=[
          pltpu.VMEM((2, K), jnp.int32),
          pltpu.VMEM((2, K), jnp.int32),
          pltpu.VMEM((2, K, s), jnp.float32),
          pltpu.VMEM((ZR, s), jnp.float32),
          pltpu.VMEM_SHARED((n_pad, s), jnp.float32),
          pltpu.SemaphoreType.DMA,
          pltpu.SemaphoreType.DMA,
      ],
  )
  def kern(msg_hbm, src_hbm, dst_hbm, out_hbm, srcb, dstb, rows, zbuf, agg,
           sem0, sem1):
    c = lax.axis_index("c")
    t = lax.axis_index("s")
    sems = (sem0, sem1)

    zero = jnp.zeros((LANES,), jnp.float32)
    for i in range(ZR):
      for j in range(s // LANES):
        zbuf[i, pl.ds(j * LANES, LANES)] = zero
    astripe = t * acc_per_tile
    for i in range(acc_per_tile // ZR):
      pltpu.sync_copy(zbuf, agg.at[pl.ds(astripe + i * ZR, ZR)])
    plsc.subcore_barrier()

    base = (c * NS + t) * per_tile

    def load_and_fire(j, b):
      off = base + j * K
      pltpu.sync_copy(src_hbm.at[pl.ds(off, K)], srcb.at[b])
      pltpu.sync_copy(dst_hbm.at[pl.ds(off, K)], dstb.at[b])
      pltpu.async_copy(msg_hbm.at[srcb.at[b]], rows.at[b], sems[b])

    def drain_and_scatter(b):
      pltpu.make_async_copy(msg_hbm.at[srcb.at[b]], rows.at[b],
                            sems[b]).wait()
      pltpu.sync_copy(rows.at[b], agg.at[dstb.at[b]], add=True)

    for b in range(min(2, chunks)):
      load_and_fire(b, b)

    def body(g, carry):
      for b in range(2):
        j = 2 * g + b
        drain_and_scatter(b)

        @pl.when(j + 2 < chunks)
        def _():
          load_and_fire(j + 2, b)

      return carry

    pairs = (chunks - 1) // 2
    lax.fori_loop(0, pairs, body, 0)
    for k in range(chunks - 2 * pairs):
      drain_and_scatter((2 * pairs + k) % 2)
    plsc.subcore_barrier()

    for i in range(acc_per_tile // ZR):
      r0 = astripe + i * ZR
      pltpu.sync_copy(agg.at[pl.ds(r0, ZR)], zbuf)
      pltpu.sync_copy(zbuf, out_hbm.at[pl.ds(c * n_pad + r0, ZR)])

  return kern(msg, src, dst)


def _sc_indeg(dst, n_pad, s):
  e = dst.shape[0]
  per_tile = e // (NC * NS)
  K = 80
  chunks = per_tile // K
  acc_per_tile = n_pad // NS
  ZR = 64

  mesh = plsc.VectorSubcoreMesh(core_axis_name="c", subcore_axis_name="s")

  @functools.partial(
      pl.kernel,
      out_type=jax.ShapeDtypeStruct((NC * n_pad, s), jnp.float32),
      mesh=mesh,
      scratch_types=[
          pltpu.VMEM((2, K), jnp.int32),
          pltpu.VMEM((K, s), jnp.float32),
          pltpu.VMEM((ZR, s), jnp.float32),
          pltpu.VMEM_SHARED((n_pad, s), jnp.float32),
          pltpu.SemaphoreType.DMA,
          pltpu.SemaphoreType.DMA,
      ],
  )
  def kern(dst_hbm, out_hbm, dstb, ones, zbuf, agg, sem0, sem1):
    c = lax.axis_index("c")
    t = lax.axis_index("s")
    sems = (sem0, sem1)

    zero = jnp.zeros((LANES,), jnp.float32)
    one = jnp.ones((LANES,), jnp.float32)
    for i in range(ZR):
      for j in range(s // LANES):
        zbuf[i, pl.ds(j * LANES, LANES)] = zero

    def fill(i, carry):
      for j in range(s // LANES):
        ones[i, pl.ds(j * LANES, LANES)] = one
      return carry

    lax.fori_loop(0, K, fill, 0)
    astripe = t * acc_per_tile
    for i in range(acc_per_tile // ZR):
      pltpu.sync_copy(zbuf, agg.at[pl.ds(astripe + i * ZR, ZR)])
    plsc.subcore_barrier()

    base = (c * NS + t) * per_tile

    def load_and_fire(j, b):
      off = base + j * K
      pltpu.sync_copy(dst_hbm.at[pl.ds(off, K)], dstb.at[b])
      pltpu.async_copy(ones, agg.at[dstb.at[b]], sems[b], add=True)

    def drain(b):
      pltpu.make_async_copy(ones, agg.at[dstb.at[b]], sems[b]).wait()

    for b in range(min(2, chunks)):
      load_and_fire(b, b)

    def body(g, carry):
      for b in range(2):
        j = 2 * g + b
        drain(b)

        @pl.when(j + 2 < chunks)
        def _():
          load_and_fire(j + 2, b)

      return carry

    pairs = (chunks - 1) // 2
    lax.fori_loop(0, pairs, body, 0)
    for k in range(chunks - 2 * pairs):
      drain((2 * pairs + k) % 2)
    plsc.subcore_barrier()

    for i in range(acc_per_tile // ZR):
      r0 = astripe + i * ZR
      pltpu.sync_copy(agg.at[pl.ds(r0, ZR)], zbuf)
      pltpu.sync_copy(zbuf, out_hbm.at[pl.ds(c * n_pad + r0, ZR)])

  return kern(dst)


_BLK = 1024


def _row_spec(s):
  return pl.BlockSpec((_BLK, s), lambda i: (i, 0))


def _full_spec(a, b):
  return pl.BlockSpec((a, b), lambda i: (0, 0))


def _dot_t(x, w):
  return lax.dot_general(x, w, (((1,), (1,)), ((), ())),
                         preferred_element_type=jnp.float32)


def _agg_specs(n, s):
  nb = n // _BLK
  return [pl.BlockSpec((_BLK, s), lambda i: (i, 0)),
          pl.BlockSpec((_BLK, s), lambda i: (i + nb, 0))]


def _tc_round0(cnt2, bm0, wu0, bu0, wm1, bm1):
  n = cnt2.shape[0] // NC
  s = wu0.shape[1]

  def body(c0_ref, c1_ref, bm0_ref, wu_ref, bu_ref, wm_ref, bm_ref,
           snew_ref, msg_ref):
    ind = c0_ref[...] + c1_ref[...]
    v0 = _dot_t(jnp.maximum(bm0_ref[...], 0.0), wu_ref[...])
    snew = jnp.maximum(ind * v0 + bu_ref[...], 0.0)
    snew_ref[...] = snew
    msg_ref[...] = jnp.maximum(_dot_t(snew, wm_ref[...]) + bm_ref[...], 0.0)

  a0_spec, a1_spec = _agg_specs(n, s)
  return pl.pallas_call(
      body,
      grid=(n // _BLK,),
      in_specs=[
          a0_spec,
          a1_spec,
          _full_spec(1, s),
          _full_spec(s, s),
          _full_spec(1, s),
          _full_spec(s, s),
          _full_spec(1, s),
      ],
      out_specs=[_row_spec(s), _row_spec(s)],
      out_shape=[jax.ShapeDtypeStruct((n, s), jnp.float32),
                 jax.ShapeDtypeStruct((n, s), jnp.float32)],
  )(cnt2, cnt2, bm0.reshape(1, s), wu0, bu0.reshape(1, s), wm1,
    bm1.reshape(1, s))


def _tc_boundary(state, agg2, wu, bu, wm, bm):
  n, s = state.shape

  def body(st_ref, a0_ref, a1_ref, wu_ref, bu_ref, wm_ref, bm_ref,
           snew_ref, msg_ref):
    agg = a0_ref[...] + a1_ref[...]
    upd = jnp.maximum(_dot_t(agg, wu_ref[...]) + bu_ref[...], 0.0)
    snew = st_ref[...] + upd
    snew_ref[...] = snew
    msg_ref[...] = jnp.maximum(_dot_t(snew, wm_ref[...]) + bm_ref[...], 0.0)

  a0_spec, a1_spec = _agg_specs(n, s)
  return pl.pallas_call(
      body,
      grid=(n // _BLK,),
      in_specs=[
          _row_spec(s),
          a0_spec,
          a1_spec,
          _full_spec(s, s),
          _full_spec(1, s),
          _full_spec(s, s),
          _full_spec(1, s),
      ],
      out_specs=[_row_spec(s), _row_spec(s)],
      out_shape=[jax.ShapeDtypeStruct((n, s), jnp.float32),
                 jax.ShapeDtypeStruct((n, s), jnp.float32)],
  )(state, agg2, agg2, wu, bu.reshape(1, s), wm, bm.reshape(1, s))


def _tc_final(state, agg2, wu, bu, we, be):
  n, s = state.shape
  two_m = we.shape[0]
  m = two_m // 2

  def body(st_ref, a0_ref, a1_ref, wu_ref, bu_ref, we_ref, be_ref, o_ref):
    agg = a0_ref[...] + a1_ref[...]
    upd = jnp.maximum(_dot_t(agg, wu_ref[...]) + bu_ref[...], 0.0)
    snew = st_ref[...] + upd
    out = _dot_t(snew, we_ref[...]) + be_ref[...]
    col = lax.broadcasted_iota(jnp.int32, (_BLK, two_m), 1)
    o_ref[...] = jnp.where(col < m, out, jnp.exp(out))

  a0_spec, a1_spec = _agg_specs(n, s)
  return pl.pallas_call(
      body,
      grid=(n // _BLK,),
      in_specs=[
          _row_spec(s),
          a0_spec,
          a1_spec,
          _full_spec(s, s),
          _full_spec(1, s),
          _full_spec(two_m, s),
          _full_spec(1, two_m),
      ],
      out_specs=_row_spec(two_m),
      out_shape=jax.ShapeDtypeStruct((n, two_m), jnp.float32),
  )(state, agg2, agg2, wu, bu.reshape(1, s), we,
    be.reshape(1, two_m))


def kernel(x, edge_index, batch, Wm, bm, Wu, bu, We, be):
  n = x.shape[0]
  s = Wm.shape[1]
  n_pad = ((n + _BLK - 1) // _BLK) * _BLK
  src = edge_index[0]
  dst = edge_index[1]

  cnt2 = _sc_indeg(dst, n_pad, s)
  state, msg = _tc_round0(cnt2, bm[0], Wu[0], bu[0], Wm[1], bm[1])
  for r in range(1, Wm.shape[0]):
    agg2 = _sc_edge_pass(msg, src, dst)
    if r + 1 < Wm.shape[0]:
      state, msg = _tc_boundary(state, agg2, Wu[r], bu[r], Wm[r + 1],
                                bm[r + 1])
    else:
      out = _tc_final(state, agg2, Wu[r], bu[r], We, be)
  return out[:n].astype(x.dtype)
